# baseline (speedup 1.0000x reference)
import functools

import jax
import jax.numpy as jnp
from jax import lax
from jax.experimental import pallas as pl
from jax.experimental.pallas import tpu as pltpu
from jax.experimental.pallas import tpu_sc as plsc

N = 10000
E = 320000
D = 128
K = 5000
NPAD = 10240
NROW = NPAD // 128
HALF = NPAD // 2
NC = 2
NS = 16
NWORK = NC * NS
EB = 80
NCH = 126
EPW = EB * NCH
EPAD = EPW * NWORK
EB2 = 128
NCHCAP = 80
CAP = NCHCAP * EB2
CAPB = CAP + 16
RPC = HALF // NS
PADIDX = NPAD - 1
SRCPAD = N
BR = 256
NBLK = NPAD // BR

_MININT = -2147483648


def _mesh():
    return plsc.VectorSubcoreMesh(core_axis_name="c", subcore_axis_name="s")


def _fill(buf, nrow, grp, val):
    v = jnp.full((16,), val, jnp.float32)

    def row(k, _):
        i = k // grp
        j = (k - i * grp) * 16
        buf[i, pl.ds(j, 16)] = v
        return 0

    lax.fori_loop(0, nrow * grp, row, 0)


def _fill_i32(buf, nrow, grp, val):
    v = jnp.full((16,), val, jnp.int32)

    def row(k, _):
        i = k // grp
        j = (k - i * grp) * 16
        buf[i, pl.ds(j, 16)] = v
        return 0

    lax.fori_loop(0, nrow * grp, row, 0)


def _ident80(ident):
    def row(k, _):
        ident[pl.ds(k * 16, 16)] = lax.iota(jnp.int32, 16) + k * 16
        return 0

    lax.fori_loop(0, NROW // 16, row, 0)


def _make_edges():
    @functools.partial(
        pl.kernel,
        out_type=(
            jax.ShapeDtypeStruct((NC * NROW, 128), jnp.float32),
            jax.ShapeDtypeStruct((NC * NROW, 128), jnp.float32),
            jax.ShapeDtypeStruct((NWORK * 4, NCHCAP, EB2), jnp.int32),
            jax.ShapeDtypeStruct((NWORK, 16), jnp.int32),
        ),
        mesh=_mesh(),
        compiler_params=pltpu.CompilerParams(needs_layout_passes=False),
        scratch_types=[
            pltpu.VMEM((NCH, EB), jnp.int32),
            pltpu.VMEM((NCH, EB), jnp.int32),
            [pltpu.VMEM((NCHCAP, EB2), jnp.int32) for _ in range(4)],
            pltpu.VMEM((NROW, 128), jnp.float32),
            pltpu.VMEM((NROW, 128), jnp.float32),
            pltpu.VMEM((NROW,), jnp.int32),
            pltpu.VMEM((16,), jnp.int32),
            pltpu.VMEM_SHARED((NROW, 128), jnp.float32),
            pltpu.VMEM_SHARED((NROW, 128), jnp.float32),
        ],
    )
    def edges(srcp, dstp, out_od, out_id, out_runs, out_cnt,
              sbuf, dbuf, comp, aod, aid, ident, cntv, sod, sid):
        c = lax.axis_index("c")
        s = lax.axis_index("s")
        wid = c * NS + s
        _fill(aod, NROW, 8, 0.0)
        _fill(aid, NROW, 8, 0.0)
        _ident80(ident)
        _fill_i32(comp[0], NCHCAP, 8, SRCPAD)
        _fill_i32(comp[1], NCHCAP, 8, 0)
        _fill_i32(comp[2], NCHCAP, 8, SRCPAD)
        _fill_i32(comp[3], NCHCAP, 8, 0)

        @pl.when(s == 0)
        def _():
            pltpu.sync_copy(aod, sod)
            pltpu.sync_copy(aid, sid)

        pltpu.sync_copy(srcp.at[wid], sbuf)
        pltpu.sync_copy(dstp.at[wid], dbuf)
        plsc.subcore_barrier()
        ones = jnp.full((16,), 1.0, jnp.float32)
        half = jnp.int32(HALF)

        def it(k, carry):
            p0, p1 = carry
            g = k // (EB // 16)
            j = (k - g * (EB // 16)) * 16
            sv = sbuf[g, pl.ds(j, 16)]
            dv = dbuf[g, pl.ds(j, 16)]
            plsc.addupdate_scatter(aod, [lax.shift_right_logical(sv, 7), sv & 127], ones)
            plsc.addupdate_scatter(aid, [lax.shift_right_logical(dv, 7), dv & 127], ones)
            m1 = dv >= half
            m0 = jnp.logical_not(m1)
            o0 = jnp.where(m0, 1, 0).astype(jnp.int32)
            o1 = jnp.where(m1, 1, 0).astype(jnp.int32)
            pos0 = plsc.cumsum(o0) - 1 + p0
            pos1 = plsc.cumsum(o1) - 1 + p1
            p0hi = lax.shift_right_logical(pos0, 7)
            p0lo = pos0 & 127
            p1hi = lax.shift_right_logical(pos1, 7)
            p1lo = pos1 & 127
            plsc.store_scatter(comp[0], [p0hi, p0lo], sv, mask=m0)
            plsc.store_scatter(comp[1], [p0hi, p0lo], dv, mask=m0)
            plsc.store_scatter(comp[2], [p1hi, p1lo], sv, mask=m1)
            plsc.store_scatter(comp[3], [p1hi, p1lo], dv - half, mask=m1)
            c0 = jnp.sum(o0)
            return (p0 + c0, p1 + (16 - c0))

        p0, p1 = lax.fori_loop(0, NCH * (EB // 16), it,
                               (jnp.int32(0), jnp.int32(0)))

        for h in range(2):
            for j in range(2):
                pltpu.sync_copy(comp[h * 2 + j],
                                out_runs.at[(wid * 2 + h) * 2 + j])
        ii = lax.iota(jnp.int32, 16)
        cntv[...] = jnp.where(ii == 0, p0, jnp.where(ii == 1, p1, 0))
        pltpu.sync_copy(cntv, out_cnt.at[wid])

        pltpu.sync_copy(aod, sod.at[ident], add=True)
        pltpu.sync_copy(aid, sid.at[ident], add=True)
        plsc.subcore_barrier()

        @pl.when(s < NROW // 8)
        def _():
            r0 = s * 8
            pltpu.sync_copy(sod.at[pl.ds(r0, 8)], out_od.at[pl.ds(c * NROW + r0, 8)])
            pltpu.sync_copy(sid.at[pl.ds(r0, 8)], out_id.at[pl.ds(c * NROW + r0, 8)])

    return edges


def _make_agg128():
    P = 4

    @functools.partial(
        pl.kernel,
        out_type=jax.ShapeDtypeStruct((NPAD, D), jnp.float32),
        mesh=_mesh(),
        compiler_params=pltpu.CompilerParams(needs_layout_passes=False),
        scratch_types=[
            [pltpu.VMEM((EB2, D), jnp.float32) for _ in range(P)],
            pltpu.VMEM((NCHCAP, EB2), jnp.int32),
            pltpu.VMEM((NCHCAP, EB2), jnp.int32),
            pltpu.VMEM((16,), jnp.int32),
            pltpu.VMEM_SHARED((HALF, D), jnp.float32),
            [pltpu.SemaphoreType.DMA for _ in range(P)],
            [pltpu.SemaphoreType.DMA for _ in range(P)],
        ],
    )
    def agg(tab, runs, cnts, out, rows, sbufr, dbufr, cntv, acc, gsems, ssems):
        c = lax.axis_index("c")
        s = lax.axis_index("s")
        _fill(rows[0], EB2, D // 16, 0.0)
        base = s * RPC
        for q in range(2):
            pltpu.sync_copy(rows[0], acc.at[pl.ds(base + q * 128, 128)])
        pltpu.sync_copy(rows[0].at[pl.ds(0, 64)], acc.at[pl.ds(base + 256, 64)])
        plsc.subcore_barrier()

        for ri in range(2):
            r = 2 * s + ri
            pltpu.sync_copy(runs.at[(r * 2 + c) * 2 + 0], sbufr)
            pltpu.sync_copy(runs.at[(r * 2 + c) * 2 + 1], dbufr)
            pltpu.sync_copy(cnts.at[r], cntv)
            cnt = jnp.sum(jnp.where(lax.iota(jnp.int32, 16) == c, cntv[...], 0))
            nch = (cnt + EB2 - 1) // EB2

            @pl.when(nch > 0)
            def _():
                pltpu.async_copy(tab.at[sbufr.at[0]], rows[0], gsems[0])

            @pl.when(nch > 1)
            def _():
                pltpu.async_copy(tab.at[sbufr.at[1]], rows[1], gsems[1])

            def rnd(R, _):
                for b in range(P):
                    g = R * P + b
                    b2 = (b + 2) % P

                    @pl.when(g < nch)
                    def _():
                        pltpu.make_async_copy(
                            tab.at[sbufr.at[g]], rows[b], gsems[b]).wait()
                        pltpu.async_copy(rows[b], acc.at[dbufr.at[g]],
                                         ssems[b], add=True)

                    @pl.when((g + 2 < nch) & (g >= P - 2))
                    def _():
                        pltpu.make_async_copy(
                            rows[b2], acc.at[dbufr.at[0]], ssems[b2]).wait()

                    @pl.when(g + 2 < nch)
                    def _():
                        pltpu.async_copy(tab.at[sbufr.at[g + 2]], rows[b2], gsems[b2])

                return 0

            lax.fori_loop(0, (nch + P - 1) // P, rnd, 0)

            for bb in range(P):
                @pl.when(nch > bb)
                def _():
                    pltpu.make_async_copy(
                        rows[bb], acc.at[dbufr.at[0]], ssems[bb]).wait()

        plsc.subcore_barrier()
        obase = c * HALF + base
        for q in range(2):
            pltpu.sync_copy(acc.at[pl.ds(base + q * 128, 128)],
                            out.at[pl.ds(obase + q * 128, 128)])
        pltpu.sync_copy(acc.at[pl.ds(base + 256, 64)],
                        out.at[pl.ds(obase + 256, 64)])

    return agg


def _make_score_agg():
    @functools.partial(
        pl.kernel,
        out_type=jax.ShapeDtypeStruct((NC * NROW, 128), jnp.float32),
        mesh=_mesh(),
        compiler_params=pltpu.CompilerParams(needs_layout_passes=False),
        scratch_types=[
            pltpu.VMEM((NCH, EB), jnp.int32),
            pltpu.VMEM((NCH, EB), jnp.int32),
            pltpu.VMEM((NROW, 128), jnp.float32),
            pltpu.VMEM((NROW, 128), jnp.float32),
            pltpu.VMEM((NROW,), jnp.int32),
            pltpu.VMEM_SHARED((NROW, 128), jnp.float32),
        ],
    )
    def sagg(t80, srcp, dstp, out, sbuf, dbuf, tbuf, acc, ident, sacc):
        c = lax.axis_index("c")
        s = lax.axis_index("s")
        wid = c * NS + s
        _fill(acc, NROW, 8, 0.0)
        _ident80(ident)

        @pl.when(s == 0)
        def _():
            pltpu.sync_copy(acc, sacc)

        pltpu.sync_copy(t80, tbuf)
        pltpu.sync_copy(srcp.at[wid], sbuf)
        pltpu.sync_copy(dstp.at[wid], dbuf)
        plsc.subcore_barrier()

        def it(k, _):
            g = k // (EB // 16)
            j = (k - g * (EB // 16)) * 16
            sv = sbuf[g, pl.ds(j, 16)]
            dv = dbuf[g, pl.ds(j, 16)]
            tv = plsc.load_gather(tbuf, [lax.shift_right_logical(sv, 7), sv & 127])
            plsc.addupdate_scatter(acc, [lax.shift_right_logical(dv, 7), dv & 127], tv)
            return 0

        lax.fori_loop(0, NCH * (EB // 16), it, 0)
        pltpu.sync_copy(acc, sacc.at[ident], add=True)
        plsc.subcore_barrier()

        @pl.when(s < NROW // 8)
        def _():
            r0 = s * 8
            pltpu.sync_copy(sacc.at[pl.ds(r0, 8)], out.at[pl.ds(c * NROW + r0, 8)])

    return sagg


def _tc_norms(od0, od1, id0, id1):
    def body(o0, o1, i0, i1, nso, ndo):
        od = o0[...] + o1[...]
        idg = i0[...] + i1[...]
        nodeid = (lax.broadcasted_iota(jnp.int32, (NROW, 128), 0) * 128
                  + lax.broadcasted_iota(jnp.int32, (NROW, 128), 1))
        valid = nodeid < N
        nso[...] = jnp.where((od > 0) & valid, lax.rsqrt(od), 0.0)
        ndo[...] = jnp.where((idg > 0) & valid, lax.rsqrt(idg), 0.0)

    return pl.pallas_call(
        body,
        out_shape=[
            jax.ShapeDtypeStruct((NROW, 128), jnp.float32),
            jax.ShapeDtypeStruct((NROW, 128), jnp.float32),
        ],
    )(od0, od1, id0, id1)


def _tc_scale(x_pad, nsc):
    def body(xr, nr, h0):
        h0[...] = xr[...] * nr[...]

    return pl.pallas_call(
        body,
        grid=(NBLK,),
        in_specs=[
            pl.BlockSpec((BR, D), lambda i: (i, 0)),
            pl.BlockSpec((BR, 1), lambda i: (i, 0)),
        ],
        out_specs=pl.BlockSpec((BR, D), lambda i: (i, 0)),
        out_shape=jax.ShapeDtypeStruct((NPAD, D), jnp.float32),
    )(x_pad, nsc)


def _tc_layer(a, ndc, nsc, W, b, Wpl):
    def body(ar, ndr, nsr, wr, br, wpr, feat, hn, scl):
        pre = ar[...] * ndr[...]
        f = jnp.dot(pre, wr[...], preferred_element_type=jnp.float32) + br[...]
        feat[...] = f
        hn[...] = f * nsr[...]
        scl[...] = jnp.dot(f, wpr[...], preferred_element_type=jnp.float32)

    return pl.pallas_call(
        body,
        grid=(NBLK,),
        in_specs=[
            pl.BlockSpec((BR, D), lambda i: (i, 0)),
            pl.BlockSpec((BR, 1), lambda i: (i, 0)),
            pl.BlockSpec((BR, 1), lambda i: (i, 0)),
            pl.BlockSpec((D, D), lambda i: (0, 0)),
            pl.BlockSpec((1, D), lambda i: (0, 0)),
            pl.BlockSpec((D, 1), lambda i: (0, 0)),
        ],
        out_specs=[
            pl.BlockSpec((BR, D), lambda i: (i, 0)),
            pl.BlockSpec((BR, D), lambda i: (i, 0)),
            pl.BlockSpec((BR, 1), lambda i: (i, 0)),
        ],
        out_shape=[
            jax.ShapeDtypeStruct((NPAD, D), jnp.float32),
            jax.ShapeDtypeStruct((NPAD, D), jnp.float32),
            jax.ShapeDtypeStruct((NPAD, 1), jnp.float32),
        ],
    )(a, ndc, nsc, W, b, Wpl)


def _tc_layer_last(a, ndc, nsc, W, b, Wpl, sc0, sc1):
    def body(ar, ndr, nsr, wr, br, wpr, s0r, s1r, feat, t):
        pre = ar[...] * ndr[...]
        f = jnp.dot(pre, wr[...], preferred_element_type=jnp.float32) + br[...]
        feat[...] = f
        stot = s0r[...] + s1r[...] + jnp.dot(f, wpr[...], preferred_element_type=jnp.float32)
        t[...] = stot * nsr[...]

    return pl.pallas_call(
        body,
        grid=(NBLK,),
        in_specs=[
            pl.BlockSpec((BR, D), lambda i: (i, 0)),
            pl.BlockSpec((BR, 1), lambda i: (i, 0)),
            pl.BlockSpec((BR, 1), lambda i: (i, 0)),
            pl.BlockSpec((D, D), lambda i: (0, 0)),
            pl.BlockSpec((1, D), lambda i: (0, 0)),
            pl.BlockSpec((D, 1), lambda i: (0, 0)),
            pl.BlockSpec((BR, 1), lambda i: (i, 0)),
            pl.BlockSpec((BR, 1), lambda i: (i, 0)),
        ],
        out_specs=[
            pl.BlockSpec((BR, D), lambda i: (i, 0)),
            pl.BlockSpec((BR, 1), lambda i: (i, 0)),
        ],
        out_shape=[
            jax.ShapeDtypeStruct((NPAD, D), jnp.float32),
            jax.ShapeDtypeStruct((NPAD, 1), jnp.float32),
        ],
    )(a, ndc, nsc, W, b, Wpl, sc0, sc1)


def _tc_select(sp0, sp1, nd80, bp11):
    def body(s0r, s1r, ndr, bpr, wout, mout):
        score = (s0r[...] + s1r[...]) * ndr[...] + bpr[...]
        ib = lax.bitcast_convert_type(score, jnp.int32)
        keyi = jnp.where(ib < 0, ib ^ jnp.int32(0x7FFFFFFF), ib)
        u = keyi ^ jnp.int32(_MININT)
        nodeid = (lax.broadcasted_iota(jnp.int32, (NROW, 128), 0) * 128
                  + lax.broadcasted_iota(jnp.int32, (NROW, 128), 1))
        valid = nodeid < N
        u = jnp.where(valid, u, 0)

        def step(it, carry):
            p, rem = carry
            i = 31 - it
            cand = p | (jnp.int32(1) << i)
            pref = lax.shift_right_logical(u, i) == lax.shift_right_logical(cand, i)
            cnt = jnp.sum(jnp.where(valid & pref, 1, 0))
            take = cnt >= rem
            return (jnp.where(take, cand, p), jnp.where(take, rem, rem - cnt))

        tsel, rem = lax.fori_loop(0, 32, step, (jnp.int32(0), jnp.int32(K)))

        tie = valid & (u == tsel)

        def tstep(it, carry):
            xl, need = carry
            i = 13 - it
            hi = xl + (jnp.int32(1) << i)
            cnt = jnp.sum(jnp.where(tie & (nodeid >= xl) & (nodeid < hi), 1, 0))
            short = cnt < need
            return (jnp.where(short, hi, xl), jnp.where(short, need - cnt, need))

        xl, _ = lax.fori_loop(0, 14, tstep, (jnp.int32(0), rem))

        tkey = tsel ^ jnp.int32(_MININT)
        sel = valid & ((keyi > tkey) | ((keyi == tkey) & (nodeid <= xl)))
        wout[...] = jnp.where(sel, jnp.tanh(score), 0.0)
        mout[...] = jnp.where(sel, 1.0, 0.0)

    return pl.pallas_call(
        body,
        out_shape=[
            jax.ShapeDtypeStruct((NROW, 128), jnp.float32),
            jax.ShapeDtypeStruct((NROW, 128), jnp.float32),
        ],
    )(sp0, sp1, nd80, bp11)


def _tc_readout(w1, m1, f0, f1, f2, M0r, gamma, beta, M1):
    def body(wr, mr, f0r, f1r, f2r, m0r, gr, br, m1r, out):
        w = wr[...]
        msk = mr[...] > 0.0
        dn = (((0,), (0,)), ((), ()))
        kf = jnp.float32(1.0 / K)
        neg = jnp.float32(-jnp.inf)
        pieces = []
        for fr in (f0r, f1r, f2r):
            pieces.append(
                lax.dot_general(w, fr[...], dn, preferred_element_type=jnp.float32) * kf)
        for fr in (f0r, f1r, f2r):
            fp = jnp.where(msk, fr[...] * w, neg)
            pieces.append(jnp.max(fp, axis=0, keepdims=True))

        hid = jnp.zeros((1, D), jnp.float32)
        for j in range(6):
            hid = hid + jnp.dot(pieces[j], m0r[j], preferred_element_type=jnp.float32)
        inv = jnp.float32(1.0 / (1.0 + 1e-5) ** 0.5)
        hb = hid * inv * gr[...] + br[...]
        hrelu = jnp.maximum(hb, 0.0)
        lo = jnp.dot(hrelu, m1r[...], preferred_element_type=jnp.float32)
        mx = jnp.max(lo, axis=-1, keepdims=True)
        ls = mx + jnp.log(jnp.sum(jnp.exp(lo - mx), axis=-1, keepdims=True))
        out[...] = lo - ls

    return pl.pallas_call(
        body,
        out_shape=jax.ShapeDtypeStruct((1, 40), jnp.float32),
    )(w1, m1, f0, f1, f2, M0r, gamma, beta, M1)


def kernel(x, edge_index, W0, b0, W1, b1, W2, b2, Wp, bp, M0, gamma, beta, M1):
    src = edge_index[0]
    dst = edge_index[1]
    pad = jnp.full((EPAD - E,), PADIDX, jnp.int32)
    srcp = jnp.concatenate([src, pad]).reshape(NWORK, NCH, EB)
    dstp = jnp.concatenate([dst, pad]).reshape(NWORK, NCH, EB)

    x_pad = jnp.pad(x, ((0, NPAD - N), (0, 0)))
    b0r = b0.reshape(1, D)
    b1r = b1.reshape(1, D)
    b2r = b2.reshape(1, D)
    wp0 = Wp[0 * D:1 * D]
    wp1 = Wp[1 * D:2 * D]
    wp2 = Wp[2 * D:3 * D]
    bp11 = bp.reshape(1, 1)
    M0r = M0.reshape(6, D, D)
    gr = gamma.reshape(1, D)
    br = beta.reshape(1, D)

    od, idg, runs, cnts = _make_edges()(srcp, dstp)

    ns80, nd80 = _tc_norms(od[:NROW], od[NROW:], idg[:NROW], idg[NROW:])
    nsc = ns80.reshape(NPAD, 1)
    ndc = nd80.reshape(NPAD, 1)

    h0 = _tc_scale(x_pad, nsc)

    agg128 = _make_agg128()

    a = agg128(h0, runs, cnts)
    feat0, h1, sc0 = _tc_layer(a, ndc, nsc, W0, b0r, wp0)
    a = agg128(h1, runs, cnts)
    feat1, h2, sc1 = _tc_layer(a, ndc, nsc, W1, b1r, wp1)
    a = agg128(h2, runs, cnts)
    feat2, t = _tc_layer_last(a, ndc, nsc, W2, b2r, wp2, sc0, sc1)

    sp = _make_score_agg()(t.reshape(NROW, 128), srcp, dstp)

    w80, m80 = _tc_select(sp[:NROW], sp[NROW:], nd80, bp11)
    return _tc_readout(w80.reshape(NPAD, 1), m80.reshape(NPAD, 1),
                       feat0, feat1, feat2, M0r, gr, br, M1)


# speedup vs baseline: 1.0171x; 1.0171x over previous
import functools

import jax
import jax.numpy as jnp
from jax import lax
from jax.experimental import pallas as pl
from jax.experimental.pallas import tpu as pltpu
from jax.experimental.pallas import tpu_sc as plsc

N = 10000
E = 320000
D = 128
K = 5000
NPAD = 10240
NROW = NPAD // 128
HALF = NPAD // 2
NC = 2
NS = 16
NWORK = NC * NS
EB = 80
NCH = 126
EPW = EB * NCH
EPAD = EPW * NWORK
EB2 = 128
NCB = 80
CAP = NCB * EB2
CAPB = CAP + 16
RPC = HALF // NS
PADIDX = NPAD - 1
SRCPAD = N
BR = 256
NBLK = NPAD // BR

_MININT = -2147483648


def _mesh():
    return plsc.VectorSubcoreMesh(core_axis_name="c", subcore_axis_name="s")


def _fill(buf, nrow, grp, val):
    v = jnp.full((16,), val, jnp.float32)

    def row(k, _):
        i = k // grp
        j = (k - i * grp) * 16
        buf[i, pl.ds(j, 16)] = v
        return 0

    lax.fori_loop(0, nrow * grp, row, 0)


def _fill_i32(buf, nrow, grp, val):
    v = jnp.full((16,), val, jnp.int32)

    def row(k, _):
        i = k // grp
        j = (k - i * grp) * 16
        buf[i, pl.ds(j, 16)] = v
        return 0

    lax.fori_loop(0, nrow * grp, row, 0)


def _ident80(ident):
    def row(k, _):
        ident[pl.ds(k * 16, 16)] = lax.iota(jnp.int32, 16) + k * 16
        return 0

    lax.fori_loop(0, NROW // 16, row, 0)


def _make_edges():
    @functools.partial(
        pl.kernel,
        out_type=(
            jax.ShapeDtypeStruct((NC * NROW, 128), jnp.float32),
            jax.ShapeDtypeStruct((NC * NROW, 128), jnp.float32),
            jax.ShapeDtypeStruct((NWORK * 4, NCB, EB2), jnp.int32),
            jax.ShapeDtypeStruct((NWORK, 16), jnp.int32),
        ),
        mesh=_mesh(),
        compiler_params=pltpu.CompilerParams(needs_layout_passes=False),
        scratch_types=[
            pltpu.VMEM((NCH, EB), jnp.int32),
            pltpu.VMEM((NCH, EB), jnp.int32),
            [pltpu.VMEM((NCB, EB2), jnp.int32) for _ in range(4)],
            pltpu.VMEM((NROW, 128), jnp.float32),
            pltpu.VMEM((NROW, 128), jnp.float32),
            pltpu.VMEM((NROW,), jnp.int32),
            pltpu.VMEM((16,), jnp.int32),
            pltpu.VMEM_SHARED((NROW, 128), jnp.float32),
            pltpu.VMEM_SHARED((NROW, 128), jnp.float32),
        ],
    )
    def edges(srcp, dstp, out_od, out_id, out_runs, out_cnt,
              sbuf, dbuf, comp, aod, aid, ident, cntv, sod, sid):
        c = lax.axis_index("c")
        s = lax.axis_index("s")
        wid = c * NS + s
        _fill(aod, NROW, 8, 0.0)
        _fill(aid, NROW, 8, 0.0)
        _ident80(ident)
        _fill_i32(comp[0], NCB, EB2 // 16, SRCPAD)
        _fill_i32(comp[1], NCB, EB2 // 16, 0)
        _fill_i32(comp[2], NCB, EB2 // 16, SRCPAD)
        _fill_i32(comp[3], NCB, EB2 // 16, 0)

        @pl.when(s == 0)
        def _():
            pltpu.sync_copy(aod, sod)
            pltpu.sync_copy(aid, sid)

        pltpu.sync_copy(srcp.at[wid], sbuf)
        pltpu.sync_copy(dstp.at[wid], dbuf)
        plsc.subcore_barrier()
        ones = jnp.full((16,), 1.0, jnp.float32)
        half = jnp.int32(HALF)

        def it(k, carry):
            p0, p1 = carry
            g = k // (EB // 16)
            j = (k - g * (EB // 16)) * 16
            sv = sbuf[g, pl.ds(j, 16)]
            dv = dbuf[g, pl.ds(j, 16)]
            plsc.addupdate_scatter(aod, [lax.shift_right_logical(sv, 7), sv & 127], ones)
            plsc.addupdate_scatter(aid, [lax.shift_right_logical(dv, 7), dv & 127], ones)
            m1 = dv >= half
            m0 = jnp.logical_not(m1)
            o0 = jnp.where(m0, 1, 0).astype(jnp.int32)
            o1 = jnp.where(m1, 1, 0).astype(jnp.int32)
            pos0 = plsc.cumsum(o0) - 1 + p0
            pos1 = plsc.cumsum(o1) - 1 + p1
            p0hi = lax.shift_right_logical(pos0, 7)
            p0lo = pos0 & 127
            p1hi = lax.shift_right_logical(pos1, 7)
            p1lo = pos1 & 127
            plsc.store_scatter(comp[0], [p0hi, p0lo], sv, mask=m0)
            plsc.store_scatter(comp[1], [p0hi, p0lo], dv, mask=m0)
            plsc.store_scatter(comp[2], [p1hi, p1lo], sv, mask=m1)
            plsc.store_scatter(comp[3], [p1hi, p1lo], dv - half, mask=m1)
            c0 = jnp.sum(o0)
            return (p0 + c0, p1 + (16 - c0))

        p0, p1 = lax.fori_loop(0, NCH * (EB // 16), it,
                               (jnp.int32(0), jnp.int32(0)))

        for h in range(2):
            for j in range(2):
                pltpu.sync_copy(comp[h * 2 + j],
                                out_runs.at[(wid * 2 + h) * 2 + j])
        ii = lax.iota(jnp.int32, 16)
        cntv[...] = jnp.where(ii == 0, p0, jnp.where(ii == 1, p1, 0))
        pltpu.sync_copy(cntv, out_cnt.at[wid])

        pltpu.sync_copy(aod, sod.at[ident], add=True)
        pltpu.sync_copy(aid, sid.at[ident], add=True)
        plsc.subcore_barrier()

        @pl.when(s < NROW // 8)
        def _():
            r0 = s * 8
            pltpu.sync_copy(sod.at[pl.ds(r0, 8)], out_od.at[pl.ds(c * NROW + r0, 8)])
            pltpu.sync_copy(sid.at[pl.ds(r0, 8)], out_id.at[pl.ds(c * NROW + r0, 8)])

    return edges


def _make_agg128():
    P = 3

    @functools.partial(
        pl.kernel,
        out_type=jax.ShapeDtypeStruct((NPAD, D), jnp.float32),
        mesh=_mesh(),
        compiler_params=pltpu.CompilerParams(needs_layout_passes=False),
        scratch_types=[
            [pltpu.VMEM((EB2, D), jnp.float32) for _ in range(P)],
            pltpu.VMEM((NCB, EB2), jnp.int32),
            pltpu.VMEM((NCB, EB2), jnp.int32),
            pltpu.VMEM((16,), jnp.int32),
            pltpu.VMEM_SHARED((HALF, D), jnp.float32),
            [pltpu.SemaphoreType.DMA for _ in range(P)],
            [pltpu.SemaphoreType.DMA for _ in range(P)],
        ],
    )
    def agg(tab, runs, cnts, out, rows, sbufr, dbufr, cntv, acc, gsems, ssems):
        c = lax.axis_index("c")
        s = lax.axis_index("s")
        _fill(rows[0], EB2, D // 16, 0.0)
        base = s * RPC
        for q in range(2):
            pltpu.sync_copy(rows[0], acc.at[pl.ds(base + q * 128, 128)])
        pltpu.sync_copy(rows[0].at[pl.ds(0, 64)], acc.at[pl.ds(base + 256, 64)])
        plsc.subcore_barrier()

        for ri in range(2):
            r = 2 * s + ri
            pltpu.sync_copy(runs.at[(r * 2 + c) * 2 + 0], sbufr)
            pltpu.sync_copy(runs.at[(r * 2 + c) * 2 + 1], dbufr)
            pltpu.sync_copy(cnts.at[r], cntv)
            cnt = jnp.sum(jnp.where(lax.iota(jnp.int32, 16) == c, cntv[...], 0))
            nch = (cnt + EB2 - 1) // EB2

            @pl.when(nch > 0)
            def _():
                pltpu.async_copy(tab.at[sbufr.at[0]], rows[0], gsems[0])

            @pl.when(nch > 1)
            def _():
                pltpu.async_copy(tab.at[sbufr.at[1]], rows[1], gsems[1])

            def rnd(R, _):
                for b in range(P):
                    g = R * P + b
                    b2 = (b + 2) % P

                    @pl.when(g < nch)
                    def _():
                        pltpu.make_async_copy(
                            tab.at[sbufr.at[g]], rows[b], gsems[b]).wait()
                        pltpu.async_copy(rows[b], acc.at[dbufr.at[g]],
                                         ssems[b], add=True)

                    @pl.when((g + 2 < nch) & (g >= P - 2))
                    def _():
                        pltpu.make_async_copy(
                            rows[b2], acc.at[dbufr.at[0]], ssems[b2]).wait()

                    @pl.when(g + 2 < nch)
                    def _():
                        pltpu.async_copy(tab.at[sbufr.at[g + 2]], rows[b2], gsems[b2])

                return 0

            lax.fori_loop(0, (nch + P - 1) // P, rnd, 0)

            for bb in range(P):
                @pl.when(nch > bb)
                def _():
                    pltpu.make_async_copy(
                        rows[bb], acc.at[dbufr.at[0]], ssems[bb]).wait()

        plsc.subcore_barrier()
        obase = c * HALF + base
        for q in range(2):
            pltpu.sync_copy(acc.at[pl.ds(base + q * 128, 128)],
                            out.at[pl.ds(obase + q * 128, 128)])
        pltpu.sync_copy(acc.at[pl.ds(base + 256, 64)],
                        out.at[pl.ds(obase + 256, 64)])

    return agg


def _make_score_agg():
    @functools.partial(
        pl.kernel,
        out_type=jax.ShapeDtypeStruct((NC * NROW, 128), jnp.float32),
        mesh=_mesh(),
        compiler_params=pltpu.CompilerParams(needs_layout_passes=False),
        scratch_types=[
            pltpu.VMEM((NCH, EB), jnp.int32),
            pltpu.VMEM((NCH, EB), jnp.int32),
            pltpu.VMEM((NROW, 128), jnp.float32),
            pltpu.VMEM((NROW, 128), jnp.float32),
            pltpu.VMEM((NROW,), jnp.int32),
            pltpu.VMEM_SHARED((NROW, 128), jnp.float32),
        ],
    )
    def sagg(t80, srcp, dstp, out, sbuf, dbuf, tbuf, acc, ident, sacc):
        c = lax.axis_index("c")
        s = lax.axis_index("s")
        wid = c * NS + s
        _fill(acc, NROW, 8, 0.0)
        _ident80(ident)

        @pl.when(s == 0)
        def _():
            pltpu.sync_copy(acc, sacc)

        pltpu.sync_copy(t80, tbuf)
        pltpu.sync_copy(srcp.at[wid], sbuf)
        pltpu.sync_copy(dstp.at[wid], dbuf)
        plsc.subcore_barrier()

        def it(k, _):
            g = k // (EB // 16)
            j = (k - g * (EB // 16)) * 16
            sv = sbuf[g, pl.ds(j, 16)]
            dv = dbuf[g, pl.ds(j, 16)]
            tv = plsc.load_gather(tbuf, [lax.shift_right_logical(sv, 7), sv & 127])
            plsc.addupdate_scatter(acc, [lax.shift_right_logical(dv, 7), dv & 127], tv)
            return 0

        lax.fori_loop(0, NCH * (EB // 16), it, 0)
        pltpu.sync_copy(acc, sacc.at[ident], add=True)
        plsc.subcore_barrier()

        @pl.when(s < NROW // 8)
        def _():
            r0 = s * 8
            pltpu.sync_copy(sacc.at[pl.ds(r0, 8)], out.at[pl.ds(c * NROW + r0, 8)])

    return sagg


def _tc_norms(od0, od1, id0, id1):
    def body(o0, o1, i0, i1, nso, ndo):
        od = o0[...] + o1[...]
        idg = i0[...] + i1[...]
        nodeid = (lax.broadcasted_iota(jnp.int32, (NROW, 128), 0) * 128
                  + lax.broadcasted_iota(jnp.int32, (NROW, 128), 1))
        valid = nodeid < N
        nso[...] = jnp.where((od > 0) & valid, lax.rsqrt(od), 0.0)
        ndo[...] = jnp.where((idg > 0) & valid, lax.rsqrt(idg), 0.0)

    return pl.pallas_call(
        body,
        out_shape=[
            jax.ShapeDtypeStruct((NROW, 128), jnp.float32),
            jax.ShapeDtypeStruct((NROW, 128), jnp.float32),
        ],
    )(od0, od1, id0, id1)


def _tc_scale(x_pad, nsc):
    def body(xr, nr, h0):
        h0[...] = xr[...] * nr[...]

    return pl.pallas_call(
        body,
        grid=(NBLK,),
        in_specs=[
            pl.BlockSpec((BR, D), lambda i: (i, 0)),
            pl.BlockSpec((BR, 1), lambda i: (i, 0)),
        ],
        out_specs=pl.BlockSpec((BR, D), lambda i: (i, 0)),
        out_shape=jax.ShapeDtypeStruct((NPAD, D), jnp.float32),
    )(x_pad, nsc)


def _tc_layer(a, ndc, nsc, W, b, Wpl):
    def body(ar, ndr, nsr, wr, br, wpr, feat, hn, scl):
        pre = ar[...] * ndr[...]
        f = jnp.dot(pre, wr[...], preferred_element_type=jnp.float32) + br[...]
        feat[...] = f
        hn[...] = f * nsr[...]
        scl[...] = jnp.dot(f, wpr[...], preferred_element_type=jnp.float32)

    return pl.pallas_call(
        body,
        grid=(NBLK,),
        in_specs=[
            pl.BlockSpec((BR, D), lambda i: (i, 0)),
            pl.BlockSpec((BR, 1), lambda i: (i, 0)),
            pl.BlockSpec((BR, 1), lambda i: (i, 0)),
            pl.BlockSpec((D, D), lambda i: (0, 0)),
            pl.BlockSpec((1, D), lambda i: (0, 0)),
            pl.BlockSpec((D, 1), lambda i: (0, 0)),
        ],
        out_specs=[
            pl.BlockSpec((BR, D), lambda i: (i, 0)),
            pl.BlockSpec((BR, D), lambda i: (i, 0)),
            pl.BlockSpec((BR, 1), lambda i: (i, 0)),
        ],
        out_shape=[
            jax.ShapeDtypeStruct((NPAD, D), jnp.float32),
            jax.ShapeDtypeStruct((NPAD, D), jnp.float32),
            jax.ShapeDtypeStruct((NPAD, 1), jnp.float32),
        ],
    )(a, ndc, nsc, W, b, Wpl)


def _tc_layer_last(a, ndc, nsc, W, b, Wpl, sc0, sc1):
    def body(ar, ndr, nsr, wr, br, wpr, s0r, s1r, feat, t):
        pre = ar[...] * ndr[...]
        f = jnp.dot(pre, wr[...], preferred_element_type=jnp.float32) + br[...]
        feat[...] = f
        stot = s0r[...] + s1r[...] + jnp.dot(f, wpr[...], preferred_element_type=jnp.float32)
        t[...] = stot * nsr[...]

    return pl.pallas_call(
        body,
        grid=(NBLK,),
        in_specs=[
            pl.BlockSpec((BR, D), lambda i: (i, 0)),
            pl.BlockSpec((BR, 1), lambda i: (i, 0)),
            pl.BlockSpec((BR, 1), lambda i: (i, 0)),
            pl.BlockSpec((D, D), lambda i: (0, 0)),
            pl.BlockSpec((1, D), lambda i: (0, 0)),
            pl.BlockSpec((D, 1), lambda i: (0, 0)),
            pl.BlockSpec((BR, 1), lambda i: (i, 0)),
            pl.BlockSpec((BR, 1), lambda i: (i, 0)),
        ],
        out_specs=[
            pl.BlockSpec((BR, D), lambda i: (i, 0)),
            pl.BlockSpec((BR, 1), lambda i: (i, 0)),
        ],
        out_shape=[
            jax.ShapeDtypeStruct((NPAD, D), jnp.float32),
            jax.ShapeDtypeStruct((NPAD, 1), jnp.float32),
        ],
    )(a, ndc, nsc, W, b, Wpl, sc0, sc1)


def _tc_select(sp0, sp1, nd80, bp11):
    def body(s0r, s1r, ndr, bpr, wout, mout):
        score = (s0r[...] + s1r[...]) * ndr[...] + bpr[...]
        ib = lax.bitcast_convert_type(score, jnp.int32)
        keyi = jnp.where(ib < 0, ib ^ jnp.int32(0x7FFFFFFF), ib)
        u = keyi ^ jnp.int32(_MININT)
        nodeid = (lax.broadcasted_iota(jnp.int32, (NROW, 128), 0) * 128
                  + lax.broadcasted_iota(jnp.int32, (NROW, 128), 1))
        valid = nodeid < N
        u = jnp.where(valid, u, 0)

        def step(it, carry):
            p, rem = carry
            i = 31 - it
            cand = p | (jnp.int32(1) << i)
            pref = lax.shift_right_logical(u, i) == lax.shift_right_logical(cand, i)
            cnt = jnp.sum(jnp.where(valid & pref, 1, 0))
            take = cnt >= rem
            return (jnp.where(take, cand, p), jnp.where(take, rem, rem - cnt))

        tsel, rem = lax.fori_loop(0, 32, step, (jnp.int32(0), jnp.int32(K)))

        tie = valid & (u == tsel)

        def tstep(it, carry):
            xl, need = carry
            i = 13 - it
            hi = xl + (jnp.int32(1) << i)
            cnt = jnp.sum(jnp.where(tie & (nodeid >= xl) & (nodeid < hi), 1, 0))
            short = cnt < need
            return (jnp.where(short, hi, xl), jnp.where(short, need - cnt, need))

        xl, _ = lax.fori_loop(0, 14, tstep, (jnp.int32(0), rem))

        tkey = tsel ^ jnp.int32(_MININT)
        sel = valid & ((keyi > tkey) | ((keyi == tkey) & (nodeid <= xl)))
        wout[...] = jnp.where(sel, jnp.tanh(score), 0.0)
        mout[...] = jnp.where(sel, 1.0, 0.0)

    return pl.pallas_call(
        body,
        out_shape=[
            jax.ShapeDtypeStruct((NROW, 128), jnp.float32),
            jax.ShapeDtypeStruct((NROW, 128), jnp.float32),
        ],
    )(sp0, sp1, nd80, bp11)


def _tc_readout(w1, m1, f0, f1, f2, M0r, gamma, beta, M1):
    def body(wr, mr, f0r, f1r, f2r, m0r, gr, br, m1r, out):
        w = wr[...]
        msk = mr[...] > 0.0
        dn = (((0,), (0,)), ((), ()))
        kf = jnp.float32(1.0 / K)
        neg = jnp.float32(-jnp.inf)
        pieces = []
        for fr in (f0r, f1r, f2r):
            pieces.append(
                lax.dot_general(w, fr[...], dn, preferred_element_type=jnp.float32) * kf)
        for fr in (f0r, f1r, f2r):
            fp = jnp.where(msk, fr[...] * w, neg)
            pieces.append(jnp.max(fp, axis=0, keepdims=True))

        hid = jnp.zeros((1, D), jnp.float32)
        for j in range(6):
            hid = hid + jnp.dot(pieces[j], m0r[j], preferred_element_type=jnp.float32)
        inv = jnp.float32(1.0 / (1.0 + 1e-5) ** 0.5)
        hb = hid * inv * gr[...] + br[...]
        hrelu = jnp.maximum(hb, 0.0)
        lo = jnp.dot(hrelu, m1r[...], preferred_element_type=jnp.float32)
        mx = jnp.max(lo, axis=-1, keepdims=True)
        ls = mx + jnp.log(jnp.sum(jnp.exp(lo - mx), axis=-1, keepdims=True))
        out[...] = lo - ls

    return pl.pallas_call(
        body,
        out_shape=jax.ShapeDtypeStruct((1, 40), jnp.float32),
    )(w1, m1, f0, f1, f2, M0r, gamma, beta, M1)


def kernel(x, edge_index, W0, b0, W1, b1, W2, b2, Wp, bp, M0, gamma, beta, M1):
    src = edge_index[0]
    dst = edge_index[1]
    pad = jnp.full((EPAD - E,), PADIDX, jnp.int32)
    srcp = jnp.concatenate([src, pad]).reshape(NWORK, NCH, EB)
    dstp = jnp.concatenate([dst, pad]).reshape(NWORK, NCH, EB)

    x_pad = jnp.pad(x, ((0, NPAD - N), (0, 0)))
    b0r = b0.reshape(1, D)
    b1r = b1.reshape(1, D)
    b2r = b2.reshape(1, D)
    wp0 = Wp[0 * D:1 * D]
    wp1 = Wp[1 * D:2 * D]
    wp2 = Wp[2 * D:3 * D]
    bp11 = bp.reshape(1, 1)
    M0r = M0.reshape(6, D, D)
    gr = gamma.reshape(1, D)
    br = beta.reshape(1, D)

    od, idg, runs, cnts = _make_edges()(srcp, dstp)

    ns80, nd80 = _tc_norms(od[:NROW], od[NROW:], idg[:NROW], idg[NROW:])
    nsc = ns80.reshape(NPAD, 1)
    ndc = nd80.reshape(NPAD, 1)

    h0 = _tc_scale(x_pad, nsc)

    agg128 = _make_agg128()

    a = agg128(h0, runs, cnts)
    feat0, h1, sc0 = _tc_layer(a, ndc, nsc, W0, b0r, wp0)
    a = agg128(h1, runs, cnts)
    feat1, h2, sc1 = _tc_layer(a, ndc, nsc, W1, b1r, wp1)
    a = agg128(h2, runs, cnts)
    feat2, t = _tc_layer_last(a, ndc, nsc, W2, b2r, wp2, sc0, sc1)

    sp = _make_score_agg()(t.reshape(NROW, 128), srcp, dstp)

    w80, m80 = _tc_select(sp[:NROW], sp[NROW:], nd80, bp11)
    return _tc_readout(w80.reshape(NPAD, 1), m80.reshape(NPAD, 1),
                       feat0, feat1, feat2, M0r, gr, br, M1)


# speedup vs baseline: 1.0182x; 1.0011x over previous
import functools

import jax
import jax.numpy as jnp
from jax import lax
from jax.experimental import pallas as pl
from jax.experimental.pallas import tpu as pltpu
from jax.experimental.pallas import tpu_sc as plsc

N = 10000
E = 320000
D = 128
K = 5000
NPAD = 10240
NROW = NPAD // 128
HALF = NPAD // 2
NC = 2
NS = 16
NWORK = NC * NS
EB = 80
NCH = 126
EPW = EB * NCH
EPAD = EPW * NWORK
EB2 = 128
NCB = 80
CAP = NCB * EB2
RPC = HALF // NS
PADIDX = NPAD - 1
SRCPAD = N
BR = 256
NBLK = NPAD // BR

_MININT = -2147483648


def _mesh():
    return plsc.VectorSubcoreMesh(core_axis_name="c", subcore_axis_name="s")


def _fill(buf, nrow, grp, val):
    v = jnp.full((16,), val, jnp.float32)

    def row(k, _):
        i = k // grp
        j = (k - i * grp) * 16
        buf[i, pl.ds(j, 16)] = v
        return 0

    lax.fori_loop(0, nrow * grp, row, 0)


def _fill_i32(buf, nrow, grp, val):
    v = jnp.full((16,), val, jnp.int32)

    def row(k, _):
        i = k // grp
        j = (k - i * grp) * 16
        buf[i, pl.ds(j, 16)] = v
        return 0

    lax.fori_loop(0, nrow * grp, row, 0)


def _ident80(ident):
    def row(k, _):
        ident[pl.ds(k * 16, 16)] = lax.iota(jnp.int32, 16) + k * 16
        return 0

    lax.fori_loop(0, NROW // 16, row, 0)


def _make_edges():
    @functools.partial(
        pl.kernel,
        out_type=(
            jax.ShapeDtypeStruct((NC * NROW, 128), jnp.float32),
            jax.ShapeDtypeStruct((NC * NROW, 128), jnp.float32),
            jax.ShapeDtypeStruct((NWORK * 4, NCB, EB2), jnp.int32),
            jax.ShapeDtypeStruct((NWORK, 16), jnp.int32),
        ),
        mesh=_mesh(),
        compiler_params=pltpu.CompilerParams(needs_layout_passes=False),
        scratch_types=[
            pltpu.VMEM((NCH, EB), jnp.int32),
            pltpu.VMEM((NCH, EB), jnp.int32),
            [pltpu.VMEM((NCB, EB2), jnp.int32) for _ in range(4)],
            pltpu.VMEM((NROW, 128), jnp.float32),
            pltpu.VMEM((NROW, 128), jnp.float32),
            pltpu.VMEM((NROW,), jnp.int32),
            pltpu.VMEM((16,), jnp.int32),
            pltpu.VMEM_SHARED((NROW, 128), jnp.float32),
            pltpu.VMEM_SHARED((NROW, 128), jnp.float32),
        ],
    )
    def edges(srcp, dstp, out_od, out_id, out_runs, out_cnt,
              sbuf, dbuf, comp, aod, aid, ident, cntv, sod, sid):
        c = lax.axis_index("c")
        s = lax.axis_index("s")
        wid = c * NS + s
        _fill(aod, NROW, 8, 0.0)
        _fill(aid, NROW, 8, 0.0)
        _ident80(ident)
        _fill_i32(comp[0], NCB, EB2 // 16, SRCPAD)
        _fill_i32(comp[1], NCB, EB2 // 16, 0)
        _fill_i32(comp[2], NCB, EB2 // 16, SRCPAD)
        _fill_i32(comp[3], NCB, EB2 // 16, 0)

        @pl.when(s == 0)
        def _():
            pltpu.sync_copy(aod, sod)
            pltpu.sync_copy(aid, sid)

        pltpu.sync_copy(srcp.at[wid], sbuf)
        pltpu.sync_copy(dstp.at[wid], dbuf)
        plsc.subcore_barrier()
        ones = jnp.full((16,), 1.0, jnp.float32)
        half = jnp.int32(HALF)

        def it(k, carry):
            p0, p1 = carry
            g = k // (EB // 16)
            j = (k - g * (EB // 16)) * 16
            sv = sbuf[g, pl.ds(j, 16)]
            dv = dbuf[g, pl.ds(j, 16)]
            plsc.addupdate_scatter(aod, [lax.shift_right_logical(sv, 7), sv & 127], ones)
            plsc.addupdate_scatter(aid, [lax.shift_right_logical(dv, 7), dv & 127], ones)
            m1 = dv >= half
            m0 = jnp.logical_not(m1)
            o0 = jnp.where(m0, 1, 0).astype(jnp.int32)
            o1 = jnp.where(m1, 1, 0).astype(jnp.int32)
            pos0 = plsc.cumsum(o0) - 1 + p0
            pos1 = plsc.cumsum(o1) - 1 + p1
            p0hi = lax.shift_right_logical(pos0, 7)
            p0lo = pos0 & 127
            p1hi = lax.shift_right_logical(pos1, 7)
            p1lo = pos1 & 127
            plsc.store_scatter(comp[0], [p0hi, p0lo], sv, mask=m0)
            plsc.store_scatter(comp[1], [p0hi, p0lo], dv, mask=m0)
            plsc.store_scatter(comp[2], [p1hi, p1lo], sv, mask=m1)
            plsc.store_scatter(comp[3], [p1hi, p1lo], dv - half, mask=m1)
            c0 = jnp.sum(o0)
            return (p0 + c0, p1 + (16 - c0))

        p0, p1 = lax.fori_loop(0, NCH * (EB // 16), it,
                               (jnp.int32(0), jnp.int32(0)))

        for h in range(2):
            for j in range(2):
                pltpu.sync_copy(comp[h * 2 + j],
                                out_runs.at[(wid * 2 + h) * 2 + j])
        ii = lax.iota(jnp.int32, 16)
        cntv[...] = jnp.where(ii == 0, p0, jnp.where(ii == 1, p1, 0))
        pltpu.sync_copy(cntv, out_cnt.at[wid])

        pltpu.sync_copy(aod, sod.at[ident], add=True)
        pltpu.sync_copy(aid, sid.at[ident], add=True)
        plsc.subcore_barrier()

        @pl.when(s < NROW // 8)
        def _():
            r0 = s * 8
            pltpu.sync_copy(sod.at[pl.ds(r0, 8)], out_od.at[pl.ds(c * NROW + r0, 8)])
            pltpu.sync_copy(sid.at[pl.ds(r0, 8)], out_id.at[pl.ds(c * NROW + r0, 8)])

    return edges


def _make_agg128():
    P = 3

    @functools.partial(
        pl.kernel,
        out_type=jax.ShapeDtypeStruct((NPAD, D), jnp.float32),
        mesh=_mesh(),
        compiler_params=pltpu.CompilerParams(needs_layout_passes=False),
        scratch_types=[
            [pltpu.VMEM((EB2, D), jnp.float32) for _ in range(P)],
            pltpu.VMEM((NCB, EB2), jnp.int32),
            pltpu.VMEM((NCB, EB2), jnp.int32),
            pltpu.VMEM((16,), jnp.int32),
            pltpu.VMEM_SHARED((HALF, D), jnp.float32),
            [pltpu.SemaphoreType.DMA for _ in range(P)],
            [pltpu.SemaphoreType.DMA for _ in range(P)],
        ],
    )
    def agg(tab, runs, cnts, out, rows, sbufr, dbufr, cntv, acc, gsems, ssems):
        c = lax.axis_index("c")
        s = lax.axis_index("s")
        _fill(rows[0], EB2, D // 16, 0.0)
        base = s * RPC
        for q in range(2):
            pltpu.sync_copy(rows[0], acc.at[pl.ds(base + q * 128, 128)])
        pltpu.sync_copy(rows[0].at[pl.ds(0, 64)], acc.at[pl.ds(base + 256, 64)])
        plsc.subcore_barrier()

        for ri in range(2):
            r = 2 * s + ri
            pltpu.sync_copy(runs.at[(r * 2 + c) * 2 + 0], sbufr)
            pltpu.sync_copy(runs.at[(r * 2 + c) * 2 + 1], dbufr)
            pltpu.sync_copy(cnts.at[r], cntv)
            cnt = jnp.sum(jnp.where(lax.iota(jnp.int32, 16) == c, cntv[...], 0))
            nch = (cnt + EB2 - 1) // EB2

            @pl.when(nch > 0)
            def _():
                pltpu.async_copy(tab.at[sbufr.at[0]], rows[0], gsems[0])

            @pl.when(nch > 1)
            def _():
                pltpu.async_copy(tab.at[sbufr.at[1]], rows[1], gsems[1])

            def rnd(R, _):
                for b in range(P):
                    g = R * P + b
                    b2 = (b + 2) % P

                    @pl.when(g < nch)
                    def _():
                        pltpu.make_async_copy(
                            tab.at[sbufr.at[g]], rows[b], gsems[b]).wait()
                        pltpu.async_copy(rows[b], acc.at[dbufr.at[g]],
                                         ssems[b], add=True)

                    @pl.when((g + 2 < nch) & (g >= P - 2))
                    def _():
                        pltpu.make_async_copy(
                            rows[b2], acc.at[dbufr.at[0]], ssems[b2]).wait()

                    @pl.when(g + 2 < nch)
                    def _():
                        pltpu.async_copy(tab.at[sbufr.at[g + 2]], rows[b2], gsems[b2])

                return 0

            lax.fori_loop(0, (nch + P - 1) // P, rnd, 0)

            for bb in range(P):
                @pl.when(nch > bb)
                def _():
                    pltpu.make_async_copy(
                        rows[bb], acc.at[dbufr.at[0]], ssems[bb]).wait()

        plsc.subcore_barrier()
        obase = c * HALF + base
        for q in range(2):
            pltpu.sync_copy(acc.at[pl.ds(base + q * 128, 128)],
                            out.at[pl.ds(obase + q * 128, 128)])
        pltpu.sync_copy(acc.at[pl.ds(base + 256, 64)],
                        out.at[pl.ds(obase + 256, 64)])

    return agg


def _make_score_agg():
    @functools.partial(
        pl.kernel,
        out_type=jax.ShapeDtypeStruct((NC * NROW, 128), jnp.float32),
        mesh=_mesh(),
        compiler_params=pltpu.CompilerParams(needs_layout_passes=False),
        scratch_types=[
            pltpu.VMEM((NCH, EB), jnp.int32),
            pltpu.VMEM((NCH, EB), jnp.int32),
            pltpu.VMEM((NROW, 128), jnp.float32),
            pltpu.VMEM((NROW, 128), jnp.float32),
            pltpu.VMEM((NROW,), jnp.int32),
            pltpu.VMEM_SHARED((NROW, 128), jnp.float32),
        ],
    )
    def sagg(t80, srcp, dstp, out, sbuf, dbuf, tbuf, acc, ident, sacc):
        c = lax.axis_index("c")
        s = lax.axis_index("s")
        wid = c * NS + s
        _fill(acc, NROW, 8, 0.0)
        _ident80(ident)

        @pl.when(s == 0)
        def _():
            pltpu.sync_copy(acc, sacc)

        pltpu.sync_copy(t80, tbuf)
        pltpu.sync_copy(srcp.at[wid], sbuf)
        pltpu.sync_copy(dstp.at[wid], dbuf)
        plsc.subcore_barrier()

        def it(k, _):
            g = k // (EB // 16)
            j = (k - g * (EB // 16)) * 16
            sv = sbuf[g, pl.ds(j, 16)]
            dv = dbuf[g, pl.ds(j, 16)]
            tv = plsc.load_gather(tbuf, [lax.shift_right_logical(sv, 7), sv & 127])
            plsc.addupdate_scatter(acc, [lax.shift_right_logical(dv, 7), dv & 127], tv)
            return 0

        lax.fori_loop(0, NCH * (EB // 16), it, 0)
        pltpu.sync_copy(acc, sacc.at[ident], add=True)
        plsc.subcore_barrier()

        @pl.when(s < NROW // 8)
        def _():
            r0 = s * 8
            pltpu.sync_copy(sacc.at[pl.ds(r0, 8)], out.at[pl.ds(c * NROW + r0, 8)])

    return sagg


def _tc_norms(od0, od1, id0, id1):
    def body(o0, o1, i0, i1, nso, ndo):
        od = o0[...] + o1[...]
        idg = i0[...] + i1[...]
        nodeid = (lax.broadcasted_iota(jnp.int32, (NROW, 128), 0) * 128
                  + lax.broadcasted_iota(jnp.int32, (NROW, 128), 1))
        valid = nodeid < N
        nso[...] = jnp.where((od > 0) & valid, lax.rsqrt(od), 0.0)
        ndo[...] = jnp.where((idg > 0) & valid, lax.rsqrt(idg), 0.0)

    return pl.pallas_call(
        body,
        out_shape=[
            jax.ShapeDtypeStruct((NROW, 128), jnp.float32),
            jax.ShapeDtypeStruct((NROW, 128), jnp.float32),
        ],
    )(od0, od1, id0, id1)


def _tc_scale(x_pad, nsc):
    def body(xr, nr, h0):
        h0[...] = xr[...] * nr[...]

    return pl.pallas_call(
        body,
        grid=(NBLK,),
        in_specs=[
            pl.BlockSpec((BR, D), lambda i: (i, 0)),
            pl.BlockSpec((BR, 1), lambda i: (i, 0)),
        ],
        out_specs=pl.BlockSpec((BR, D), lambda i: (i, 0)),
        out_shape=jax.ShapeDtypeStruct((NPAD, D), jnp.float32),
    )(x_pad, nsc)


def _tc_layer(a, ndc, nsc, W, b, Wpl):
    def body(ar, ndr, nsr, wr, br, wpr, feat, hn, scl):
        pre = ar[...] * ndr[...]
        f = jnp.dot(pre, wr[...], preferred_element_type=jnp.float32) + br[...]
        feat[...] = f
        hn[...] = f * nsr[...]
        scl[...] = jnp.dot(f, wpr[...], preferred_element_type=jnp.float32)

    return pl.pallas_call(
        body,
        grid=(NBLK,),
        in_specs=[
            pl.BlockSpec((BR, D), lambda i: (i, 0)),
            pl.BlockSpec((BR, 1), lambda i: (i, 0)),
            pl.BlockSpec((BR, 1), lambda i: (i, 0)),
            pl.BlockSpec((D, D), lambda i: (0, 0)),
            pl.BlockSpec((1, D), lambda i: (0, 0)),
            pl.BlockSpec((D, 1), lambda i: (0, 0)),
        ],
        out_specs=[
            pl.BlockSpec((BR, D), lambda i: (i, 0)),
            pl.BlockSpec((BR, D), lambda i: (i, 0)),
            pl.BlockSpec((BR, 1), lambda i: (i, 0)),
        ],
        out_shape=[
            jax.ShapeDtypeStruct((NPAD, D), jnp.float32),
            jax.ShapeDtypeStruct((NPAD, D), jnp.float32),
            jax.ShapeDtypeStruct((NPAD, 1), jnp.float32),
        ],
    )(a, ndc, nsc, W, b, Wpl)


def _tc_layer_last(a, ndc, nsc, W, b, Wpl, sc0, sc1):
    def body(ar, ndr, nsr, wr, br, wpr, s0r, s1r, feat, t):
        pre = ar[...] * ndr[...]
        f = jnp.dot(pre, wr[...], preferred_element_type=jnp.float32) + br[...]
        feat[...] = f
        stot = s0r[...] + s1r[...] + jnp.dot(f, wpr[...], preferred_element_type=jnp.float32)
        t[...] = stot * nsr[...]

    return pl.pallas_call(
        body,
        grid=(NBLK,),
        in_specs=[
            pl.BlockSpec((BR, D), lambda i: (i, 0)),
            pl.BlockSpec((BR, 1), lambda i: (i, 0)),
            pl.BlockSpec((BR, 1), lambda i: (i, 0)),
            pl.BlockSpec((D, D), lambda i: (0, 0)),
            pl.BlockSpec((1, D), lambda i: (0, 0)),
            pl.BlockSpec((D, 1), lambda i: (0, 0)),
            pl.BlockSpec((BR, 1), lambda i: (i, 0)),
            pl.BlockSpec((BR, 1), lambda i: (i, 0)),
        ],
        out_specs=[
            pl.BlockSpec((BR, D), lambda i: (i, 0)),
            pl.BlockSpec((BR, 1), lambda i: (i, 0)),
        ],
        out_shape=[
            jax.ShapeDtypeStruct((NPAD, D), jnp.float32),
            jax.ShapeDtypeStruct((NPAD, 1), jnp.float32),
        ],
    )(a, ndc, nsc, W, b, Wpl, sc0, sc1)


def _tc_select(sp0, sp1, nd80, bp11):
    def body(s0r, s1r, ndr, bpr, wout, mout):
        score = (s0r[...] + s1r[...]) * ndr[...] + bpr[...]
        ib = lax.bitcast_convert_type(score, jnp.int32)
        keyi = jnp.where(ib < 0, ib ^ jnp.int32(0x7FFFFFFF), ib)
        u = keyi ^ jnp.int32(_MININT)
        nodeid = (lax.broadcasted_iota(jnp.int32, (NROW, 128), 0) * 128
                  + lax.broadcasted_iota(jnp.int32, (NROW, 128), 1))
        valid = nodeid < N
        u = jnp.where(valid, u, 0)

        def step(it, carry):
            p, rem = carry
            i = 31 - it
            cand = p | (jnp.int32(1) << i)
            pref = lax.shift_right_logical(u, i) == lax.shift_right_logical(cand, i)
            cnt = jnp.sum(jnp.where(valid & pref, 1, 0))
            take = cnt >= rem
            return (jnp.where(take, cand, p), jnp.where(take, rem, rem - cnt))

        tsel, rem = lax.fori_loop(0, 32, step, (jnp.int32(0), jnp.int32(K)))

        tie = valid & (u == tsel)

        def tstep(it, carry):
            xl, need = carry
            i = 13 - it
            hi = xl + (jnp.int32(1) << i)
            cnt = jnp.sum(jnp.where(tie & (nodeid >= xl) & (nodeid < hi), 1, 0))
            short = cnt < need
            return (jnp.where(short, hi, xl), jnp.where(short, need - cnt, need))

        xl, _ = lax.fori_loop(0, 14, tstep, (jnp.int32(0), rem))

        tkey = tsel ^ jnp.int32(_MININT)
        sel = valid & ((keyi > tkey) | ((keyi == tkey) & (nodeid <= xl)))
        wout[...] = jnp.where(sel, jnp.tanh(score), 0.0)
        mout[...] = jnp.where(sel, 1.0, 0.0)

    return pl.pallas_call(
        body,
        out_shape=[
            jax.ShapeDtypeStruct((NROW, 128), jnp.float32),
            jax.ShapeDtypeStruct((NROW, 128), jnp.float32),
        ],
    )(sp0, sp1, nd80, bp11)


def _tc_readout(w1, m1, f0, f1, f2, M0r, gamma, beta, M1):
    def body(wr, mr, f0r, f1r, f2r, m0r, gr, br, m1r, out):
        w = wr[...]
        msk = mr[...] > 0.0
        dn = (((0,), (0,)), ((), ()))
        kf = jnp.float32(1.0 / K)
        neg = jnp.float32(-jnp.inf)
        pieces = []
        for fr in (f0r, f1r, f2r):
            pieces.append(
                lax.dot_general(w, fr[...], dn, preferred_element_type=jnp.float32) * kf)
        for fr in (f0r, f1r, f2r):
            fp = jnp.where(msk, fr[...] * w, neg)
            pieces.append(jnp.max(fp, axis=0, keepdims=True))

        hid = jnp.zeros((1, D), jnp.float32)
        for j in range(6):
            hid = hid + jnp.dot(pieces[j], m0r[j], preferred_element_type=jnp.float32)
        inv = jnp.float32(1.0 / (1.0 + 1e-5) ** 0.5)
        hb = hid * inv * gr[...] + br[...]
        hrelu = jnp.maximum(hb, 0.0)
        lo = jnp.dot(hrelu, m1r[...], preferred_element_type=jnp.float32)
        mx = jnp.max(lo, axis=-1, keepdims=True)
        ls = mx + jnp.log(jnp.sum(jnp.exp(lo - mx), axis=-1, keepdims=True))
        out[...] = lo - ls

    return pl.pallas_call(
        body,
        out_shape=jax.ShapeDtypeStruct((1, 40), jnp.float32),
    )(w1, m1, f0, f1, f2, M0r, gamma, beta, M1)


def kernel(x, edge_index, W0, b0, W1, b1, W2, b2, Wp, bp, M0, gamma, beta, M1):
    src = edge_index[0]
    dst = edge_index[1]
    pad = jnp.full((EPAD - E,), PADIDX, jnp.int32)
    srcp = jnp.concatenate([src, pad]).reshape(NWORK, NCH, EB)
    dstp = jnp.concatenate([dst, pad]).reshape(NWORK, NCH, EB)

    x_pad = jnp.pad(x, ((0, NPAD - N), (0, 0)))
    b0r = b0.reshape(1, D)
    b1r = b1.reshape(1, D)
    b2r = b2.reshape(1, D)
    wp0 = Wp[0 * D:1 * D]
    wp1 = Wp[1 * D:2 * D]
    wp2 = Wp[2 * D:3 * D]
    bp11 = bp.reshape(1, 1)
    M0r = M0.reshape(6, D, D)
    gr = gamma.reshape(1, D)
    br = beta.reshape(1, D)

    od, idg, runs, cnts = _make_edges()(srcp, dstp)

    ns80, nd80 = _tc_norms(od[:NROW], od[NROW:], idg[:NROW], idg[NROW:])
    nsc = ns80.reshape(NPAD, 1)
    ndc = nd80.reshape(NPAD, 1)

    h0 = _tc_scale(x_pad, nsc)

    agg128 = _make_agg128()

    a = agg128(h0, runs, cnts)
    feat0, h1, sc0 = _tc_layer(a, ndc, nsc, W0, b0r, wp0)
    a = agg128(h1, runs, cnts)
    feat1, h2, sc1 = _tc_layer(a, ndc, nsc, W1, b1r, wp1)
    a = agg128(h2, runs, cnts)
    feat2, t = _tc_layer_last(a, ndc, nsc, W2, b2r, wp2, sc0, sc1)

    sp = _make_score_agg()(t.reshape(NROW, 128), srcp, dstp)

    w80, m80 = _tc_select(sp[:NROW], sp[NROW:], nd80, bp11)
    return _tc_readout(w80.reshape(NPAD, 1), m80.reshape(NPAD, 1),
                       feat0, feat1, feat2, M0r, gr, br, M1)


# speedup vs baseline: 1.2384x; 1.2163x over previous
import functools

import jax
import jax.numpy as jnp
from jax import lax
from jax.experimental import pallas as pl
from jax.experimental.pallas import tpu as pltpu
from jax.experimental.pallas import tpu_sc as plsc

N = 10000
E = 320000
D = 128
K = 5000
NPAD = 10240
NROW = NPAD // 128
HALF = NPAD // 2
NC = 2
NS = 16
NWORK = NC * NS
EB = 80
NCH = 125
EPW = EB * NCH
EB2 = 128
NCB = 80
CAP = NCB * EB2
RPC = HALF // NS
SRCPAD = N
BR = 256
NBLK = NPAD // BR

_MININT = -2147483648


def _mesh():
    return plsc.VectorSubcoreMesh(core_axis_name="c", subcore_axis_name="s")


def _fill(buf, nrow, grp, val):
    v = jnp.full((16,), val, jnp.float32)

    def row(k, _):
        i = k // grp
        j = (k - i * grp) * 16
        buf[i, pl.ds(j, 16)] = v
        return 0

    lax.fori_loop(0, nrow * grp, row, 0)


def _fill_i32(buf, nrow, grp, val):
    v = jnp.full((16,), val, jnp.int32)

    def row(k, _):
        i = k // grp
        j = (k - i * grp) * 16
        buf[i, pl.ds(j, 16)] = v
        return 0

    lax.fori_loop(0, nrow * grp, row, 0)


def _ident80(ident):
    def row(k, _):
        ident[pl.ds(k * 16, 16)] = lax.iota(jnp.int32, 16) + k * 16
        return 0

    lax.fori_loop(0, NROW // 16, row, 0)


def _make_edges():
    @functools.partial(
        pl.kernel,
        out_type=(
            jax.ShapeDtypeStruct((NC * NROW, 128), jnp.float32),
            jax.ShapeDtypeStruct((NC * NROW, 128), jnp.float32),
            jax.ShapeDtypeStruct((NWORK * 4, NCB, EB2), jnp.int32),
            jax.ShapeDtypeStruct((NWORK, 16), jnp.int32),
        ),
        mesh=_mesh(),
        compiler_params=pltpu.CompilerParams(needs_layout_passes=False),
        scratch_types=[
            pltpu.VMEM((NCH, EB), jnp.int32),
            pltpu.VMEM((NCH, EB), jnp.int32),
            [pltpu.VMEM((NCB, EB2), jnp.int32) for _ in range(4)],
            pltpu.VMEM((NROW, 128), jnp.float32),
            pltpu.VMEM((NROW, 128), jnp.float32),
            pltpu.VMEM((NROW,), jnp.int32),
            pltpu.VMEM((16,), jnp.int32),
            pltpu.VMEM_SHARED((NROW, 128), jnp.float32),
            pltpu.VMEM_SHARED((NROW, 128), jnp.float32),
        ],
    )
    def edges(srcp, dstp, out_od, out_id, out_runs, out_cnt,
              sbuf, dbuf, comp, aod, aid, ident, cntv, sod, sid):
        c = lax.axis_index("c")
        s = lax.axis_index("s")
        wid = c * NS + s
        _fill(aod, NROW, 8, 0.0)
        _fill(aid, NROW, 8, 0.0)
        _ident80(ident)
        _fill_i32(comp[0], NCB, EB2 // 16, SRCPAD)
        _fill_i32(comp[1], NCB, EB2 // 16, 0)
        _fill_i32(comp[2], NCB, EB2 // 16, SRCPAD)
        _fill_i32(comp[3], NCB, EB2 // 16, 0)

        @pl.when(s == 0)
        def _():
            pltpu.sync_copy(aod, sod)
            pltpu.sync_copy(aid, sid)

        pltpu.sync_copy(srcp.at[wid], sbuf)
        pltpu.sync_copy(dstp.at[wid], dbuf)
        plsc.subcore_barrier()
        ones = jnp.full((16,), 1.0, jnp.float32)
        half = jnp.int32(HALF)

        def it(k, carry):
            p0, p1 = carry
            g = k // (EB // 16)
            j = (k - g * (EB // 16)) * 16
            sv = sbuf[g, pl.ds(j, 16)]
            dv = dbuf[g, pl.ds(j, 16)]
            plsc.addupdate_scatter(aod, [lax.shift_right_logical(sv, 7), sv & 127], ones)
            plsc.addupdate_scatter(aid, [lax.shift_right_logical(dv, 7), dv & 127], ones)
            m1 = dv >= half
            m0 = jnp.logical_not(m1)
            o0 = jnp.where(m0, 1, 0).astype(jnp.int32)
            o1 = jnp.where(m1, 1, 0).astype(jnp.int32)
            pos0 = plsc.cumsum(o0) - 1 + p0
            pos1 = plsc.cumsum(o1) - 1 + p1
            p0hi = lax.shift_right_logical(pos0, 7)
            p0lo = pos0 & 127
            p1hi = lax.shift_right_logical(pos1, 7)
            p1lo = pos1 & 127
            plsc.store_scatter(comp[0], [p0hi, p0lo], sv, mask=m0)
            plsc.store_scatter(comp[1], [p0hi, p0lo], dv, mask=m0)
            plsc.store_scatter(comp[2], [p1hi, p1lo], sv, mask=m1)
            plsc.store_scatter(comp[3], [p1hi, p1lo], dv - half, mask=m1)
            c0 = jnp.sum(o0)
            return (p0 + c0, p1 + (16 - c0))

        p0, p1 = lax.fori_loop(0, NCH * (EB // 16), it,
                               (jnp.int32(0), jnp.int32(0)))

        for h in range(2):
            for j in range(2):
                pltpu.sync_copy(comp[h * 2 + j],
                                out_runs.at[(wid * 2 + h) * 2 + j])
        ii = lax.iota(jnp.int32, 16)
        cntv[...] = jnp.where(ii == 0, p0, jnp.where(ii == 1, p1, 0))
        pltpu.sync_copy(cntv, out_cnt.at[wid])

        pltpu.sync_copy(aod, sod.at[ident], add=True)
        pltpu.sync_copy(aid, sid.at[ident], add=True)
        plsc.subcore_barrier()

        @pl.when(s < NROW // 8)
        def _():
            r0 = s * 8
            pltpu.sync_copy(sod.at[pl.ds(r0, 8)], out_od.at[pl.ds(c * NROW + r0, 8)])
            pltpu.sync_copy(sid.at[pl.ds(r0, 8)], out_id.at[pl.ds(c * NROW + r0, 8)])

    return edges


def _make_agg128():
    P = 3

    @functools.partial(
        pl.kernel,
        out_type=jax.ShapeDtypeStruct((NPAD, D), jnp.float32),
        mesh=_mesh(),
        compiler_params=pltpu.CompilerParams(needs_layout_passes=False),
        scratch_types=[
            [pltpu.VMEM((EB2, D), jnp.float32) for _ in range(P)],
            pltpu.VMEM((NCB, EB2), jnp.int32),
            pltpu.VMEM((NCB, EB2), jnp.int32),
            pltpu.VMEM((16,), jnp.int32),
            pltpu.VMEM_SHARED((HALF, D), jnp.float32),
            [pltpu.SemaphoreType.DMA for _ in range(P)],
            [pltpu.SemaphoreType.DMA for _ in range(P)],
        ],
    )
    def agg(tab, runs, cnts, out, rows, sbufr, dbufr, cntv, acc, gsems, ssems):
        c = lax.axis_index("c")
        s = lax.axis_index("s")
        _fill(rows[0], EB2, D // 16, 0.0)
        base = s * RPC
        for q in range(2):
            pltpu.sync_copy(rows[0], acc.at[pl.ds(base + q * 128, 128)])
        pltpu.sync_copy(rows[0].at[pl.ds(0, 64)], acc.at[pl.ds(base + 256, 64)])
        plsc.subcore_barrier()

        for ri in range(2):
            r = 2 * s + ri
            pltpu.sync_copy(runs.at[(r * 2 + c) * 2 + 0], sbufr)
            pltpu.sync_copy(runs.at[(r * 2 + c) * 2 + 1], dbufr)
            pltpu.sync_copy(cnts.at[r], cntv)
            cnt = jnp.sum(jnp.where(lax.iota(jnp.int32, 16) == c, cntv[...], 0))
            nch = (cnt + EB2 - 1) // EB2

            @pl.when(nch > 0)
            def _():
                pltpu.async_copy(tab.at[sbufr.at[0]], rows[0], gsems[0])

            @pl.when(nch > 1)
            def _():
                pltpu.async_copy(tab.at[sbufr.at[1]], rows[1], gsems[1])

            def rnd(R, _):
                for b in range(P):
                    g = R * P + b
                    b2 = (b + 2) % P

                    @pl.when(g < nch)
                    def _():
                        pltpu.make_async_copy(
                            tab.at[sbufr.at[g]], rows[b], gsems[b]).wait()
                        pltpu.async_copy(rows[b], acc.at[dbufr.at[g]],
                                         ssems[b], add=True)

                    @pl.when((g + 2 < nch) & (g >= P - 2))
                    def _():
                        pltpu.make_async_copy(
                            rows[b2], acc.at[dbufr.at[0]], ssems[b2]).wait()

                    @pl.when(g + 2 < nch)
                    def _():
                        pltpu.async_copy(tab.at[sbufr.at[g + 2]], rows[b2], gsems[b2])

                return 0

            lax.fori_loop(0, (nch + P - 1) // P, rnd, 0)

            for bb in range(P):
                @pl.when(nch > bb)
                def _():
                    pltpu.make_async_copy(
                        rows[bb], acc.at[dbufr.at[0]], ssems[bb]).wait()

        plsc.subcore_barrier()
        obase = c * HALF + base
        for q in range(2):
            pltpu.sync_copy(acc.at[pl.ds(base + q * 128, 128)],
                            out.at[pl.ds(obase + q * 128, 128)])
        pltpu.sync_copy(acc.at[pl.ds(base + 256, 64)],
                        out.at[pl.ds(obase + 256, 64)])

    return agg


def _make_score_agg():
    @functools.partial(
        pl.kernel,
        out_type=jax.ShapeDtypeStruct((NC * NROW, 128), jnp.float32),
        mesh=_mesh(),
        compiler_params=pltpu.CompilerParams(needs_layout_passes=False),
        scratch_types=[
            pltpu.VMEM((NCH, EB), jnp.int32),
            pltpu.VMEM((NCH, EB), jnp.int32),
            pltpu.VMEM((NROW, 128), jnp.float32),
            pltpu.VMEM((NROW, 128), jnp.float32),
            pltpu.VMEM((NROW,), jnp.int32),
            pltpu.VMEM_SHARED((NROW, 128), jnp.float32),
        ],
    )
    def sagg(t80, srcp, dstp, out, sbuf, dbuf, tbuf, acc, ident, sacc):
        c = lax.axis_index("c")
        s = lax.axis_index("s")
        wid = c * NS + s
        _fill(acc, NROW, 8, 0.0)
        _ident80(ident)

        @pl.when(s == 0)
        def _():
            pltpu.sync_copy(acc, sacc)

        pltpu.sync_copy(t80, tbuf)
        pltpu.sync_copy(srcp.at[wid], sbuf)
        pltpu.sync_copy(dstp.at[wid], dbuf)
        plsc.subcore_barrier()

        def it(k, _):
            g = k // (EB // 16)
            j = (k - g * (EB // 16)) * 16
            sv = sbuf[g, pl.ds(j, 16)]
            dv = dbuf[g, pl.ds(j, 16)]
            tv = plsc.load_gather(tbuf, [lax.shift_right_logical(sv, 7), sv & 127])
            plsc.addupdate_scatter(acc, [lax.shift_right_logical(dv, 7), dv & 127], tv)
            return 0

        lax.fori_loop(0, NCH * (EB // 16), it, 0)
        pltpu.sync_copy(acc, sacc.at[ident], add=True)
        plsc.subcore_barrier()

        @pl.when(s < NROW // 8)
        def _():
            r0 = s * 8
            pltpu.sync_copy(sacc.at[pl.ds(r0, 8)], out.at[pl.ds(c * NROW + r0, 8)])

    return sagg


def _tc_norms(od0, od1, id0, id1):
    def body(o0, o1, i0, i1, nso, ndo):
        od = o0[...] + o1[...]
        idg = i0[...] + i1[...]
        nodeid = (lax.broadcasted_iota(jnp.int32, (NROW, 128), 0) * 128
                  + lax.broadcasted_iota(jnp.int32, (NROW, 128), 1))
        valid = nodeid < N
        nso[...] = jnp.where((od > 0) & valid, lax.rsqrt(od), 0.0)
        ndo[...] = jnp.where((idg > 0) & valid, lax.rsqrt(idg), 0.0)

    return pl.pallas_call(
        body,
        out_shape=[
            jax.ShapeDtypeStruct((NROW, 128), jnp.float32),
            jax.ShapeDtypeStruct((NROW, 128), jnp.float32),
        ],
    )(od0, od1, id0, id1)


def _tc_scale(x_pad, nsc):
    def body(xr, nr, h0):
        h0[...] = xr[...] * nr[...]

    return pl.pallas_call(
        body,
        grid=(NBLK,),
        in_specs=[
            pl.BlockSpec((BR, D), lambda i: (i, 0)),
            pl.BlockSpec((BR, 1), lambda i: (i, 0)),
        ],
        out_specs=pl.BlockSpec((BR, D), lambda i: (i, 0)),
        out_shape=jax.ShapeDtypeStruct((NPAD, D), jnp.float32),
    )(x_pad, nsc)


def _tc_layer(a, ndc, nsc, W, b, Wpl):
    def body(ar, ndr, nsr, wr, br, wpr, feat, hn, scl):
        pre = ar[...] * ndr[...]
        f = jnp.dot(pre, wr[...], preferred_element_type=jnp.float32) + br[...]
        feat[...] = f
        hn[...] = f * nsr[...]
        scl[...] = jnp.dot(f, wpr[...], preferred_element_type=jnp.float32)

    return pl.pallas_call(
        body,
        grid=(NBLK,),
        in_specs=[
            pl.BlockSpec((BR, D), lambda i: (i, 0)),
            pl.BlockSpec((BR, 1), lambda i: (i, 0)),
            pl.BlockSpec((BR, 1), lambda i: (i, 0)),
            pl.BlockSpec((D, D), lambda i: (0, 0)),
            pl.BlockSpec((1, D), lambda i: (0, 0)),
            pl.BlockSpec((D, 1), lambda i: (0, 0)),
        ],
        out_specs=[
            pl.BlockSpec((BR, D), lambda i: (i, 0)),
            pl.BlockSpec((BR, D), lambda i: (i, 0)),
            pl.BlockSpec((BR, 1), lambda i: (i, 0)),
        ],
        out_shape=[
            jax.ShapeDtypeStruct((NPAD, D), jnp.float32),
            jax.ShapeDtypeStruct((NPAD, D), jnp.float32),
            jax.ShapeDtypeStruct((NPAD, 1), jnp.float32),
        ],
    )(a, ndc, nsc, W, b, Wpl)


def _tc_layer_last(a, ndc, nsc, W, b, Wpl, sc0, sc1):
    def body(ar, ndr, nsr, wr, br, wpr, s0r, s1r, feat, t):
        pre = ar[...] * ndr[...]
        f = jnp.dot(pre, wr[...], preferred_element_type=jnp.float32) + br[...]
        feat[...] = f
        stot = s0r[...] + s1r[...] + jnp.dot(f, wpr[...], preferred_element_type=jnp.float32)
        t[...] = stot * nsr[...]

    return pl.pallas_call(
        body,
        grid=(NBLK,),
        in_specs=[
            pl.BlockSpec((BR, D), lambda i: (i, 0)),
            pl.BlockSpec((BR, 1), lambda i: (i, 0)),
            pl.BlockSpec((BR, 1), lambda i: (i, 0)),
            pl.BlockSpec((D, D), lambda i: (0, 0)),
            pl.BlockSpec((1, D), lambda i: (0, 0)),
            pl.BlockSpec((D, 1), lambda i: (0, 0)),
            pl.BlockSpec((BR, 1), lambda i: (i, 0)),
            pl.BlockSpec((BR, 1), lambda i: (i, 0)),
        ],
        out_specs=[
            pl.BlockSpec((BR, D), lambda i: (i, 0)),
            pl.BlockSpec((BR, 1), lambda i: (i, 0)),
        ],
        out_shape=[
            jax.ShapeDtypeStruct((NPAD, D), jnp.float32),
            jax.ShapeDtypeStruct((NPAD, 1), jnp.float32),
        ],
    )(a, ndc, nsc, W, b, Wpl, sc0, sc1)


def _tc_select(sp0, sp1, nd80, bp11):
    def body(s0r, s1r, ndr, bpr, wout, mout):
        score = (s0r[...] + s1r[...]) * ndr[...] + bpr[...]
        ib = lax.bitcast_convert_type(score, jnp.int32)
        keyi = jnp.where(ib < 0, ib ^ jnp.int32(0x7FFFFFFF), ib)
        u = keyi ^ jnp.int32(_MININT)
        nodeid = (lax.broadcasted_iota(jnp.int32, (NROW, 128), 0) * 128
                  + lax.broadcasted_iota(jnp.int32, (NROW, 128), 1))
        valid = nodeid < N
        u = jnp.where(valid, u, 0)

        def step(it, carry):
            p, rem = carry
            i = 31 - it
            cand = p | (jnp.int32(1) << i)
            pref = lax.shift_right_logical(u, i) == lax.shift_right_logical(cand, i)
            cnt = jnp.sum(jnp.where(valid & pref, 1, 0))
            take = cnt >= rem
            return (jnp.where(take, cand, p), jnp.where(take, rem, rem - cnt))

        tsel, rem = lax.fori_loop(0, 32, step, (jnp.int32(0), jnp.int32(K)))

        tie = valid & (u == tsel)

        def tstep(it, carry):
            xl, need = carry
            i = 13 - it
            hi = xl + (jnp.int32(1) << i)
            cnt = jnp.sum(jnp.where(tie & (nodeid >= xl) & (nodeid < hi), 1, 0))
            short = cnt < need
            return (jnp.where(short, hi, xl), jnp.where(short, need - cnt, need))

        xl, _ = lax.fori_loop(0, 14, tstep, (jnp.int32(0), rem))

        tkey = tsel ^ jnp.int32(_MININT)
        sel = valid & ((keyi > tkey) | ((keyi == tkey) & (nodeid <= xl)))
        wout[...] = jnp.where(sel, jnp.tanh(score), 0.0)
        mout[...] = jnp.where(sel, 1.0, 0.0)

    return pl.pallas_call(
        body,
        out_shape=[
            jax.ShapeDtypeStruct((NROW, 128), jnp.float32),
            jax.ShapeDtypeStruct((NROW, 128), jnp.float32),
        ],
    )(sp0, sp1, nd80, bp11)


def _tc_readout(w1, m1, f0, f1, f2, M0r, gamma, beta, M1):
    def body(wr, mr, f0r, f1r, f2r, m0r, gr, br, m1r, out):
        w = wr[...]
        msk = mr[...] > 0.0
        dn = (((0,), (0,)), ((), ()))
        kf = jnp.float32(1.0 / K)
        neg = jnp.float32(-jnp.inf)
        pieces = []
        for fr in (f0r, f1r, f2r):
            pieces.append(
                lax.dot_general(w, fr[...], dn, preferred_element_type=jnp.float32) * kf)
        for fr in (f0r, f1r, f2r):
            fp = jnp.where(msk, fr[...] * w, neg)
            pieces.append(jnp.max(fp, axis=0, keepdims=True))

        hid = jnp.zeros((1, D), jnp.float32)
        for j in range(6):
            hid = hid + jnp.dot(pieces[j], m0r[j], preferred_element_type=jnp.float32)
        inv = jnp.float32(1.0 / (1.0 + 1e-5) ** 0.5)
        hb = hid * inv * gr[...] + br[...]
        hrelu = jnp.maximum(hb, 0.0)
        lo = jnp.dot(hrelu, m1r[...], preferred_element_type=jnp.float32)
        mx = jnp.max(lo, axis=-1, keepdims=True)
        ls = mx + jnp.log(jnp.sum(jnp.exp(lo - mx), axis=-1, keepdims=True))
        out[...] = lo - ls

    return pl.pallas_call(
        body,
        out_shape=jax.ShapeDtypeStruct((1, 40), jnp.float32),
    )(w1, m1, f0, f1, f2, M0r, gamma, beta, M1)


def kernel(x, edge_index, W0, b0, W1, b1, W2, b2, Wp, bp, M0, gamma, beta, M1):
    srcp = edge_index[0].reshape(NWORK, NCH, EB)
    dstp = edge_index[1].reshape(NWORK, NCH, EB)

    x_pad = jnp.pad(x, ((0, NPAD - N), (0, 0)))
    b0r = b0.reshape(1, D)
    b1r = b1.reshape(1, D)
    b2r = b2.reshape(1, D)
    wp0 = Wp[0 * D:1 * D]
    wp1 = Wp[1 * D:2 * D]
    wp2 = Wp[2 * D:3 * D]
    bp11 = bp.reshape(1, 1)
    M0r = M0.reshape(6, D, D)
    gr = gamma.reshape(1, D)
    br = beta.reshape(1, D)

    od, idg, runs, cnts = _make_edges()(srcp, dstp)

    ns80, nd80 = _tc_norms(od[:NROW], od[NROW:], idg[:NROW], idg[NROW:])
    nsc = ns80.reshape(NPAD, 1)
    ndc = nd80.reshape(NPAD, 1)

    h0 = _tc_scale(x_pad, nsc)

    agg128 = _make_agg128()

    a = agg128(h0, runs, cnts)
    feat0, h1, sc0 = _tc_layer(a, ndc, nsc, W0, b0r, wp0)
    a = agg128(h1, runs, cnts)
    feat1, h2, sc1 = _tc_layer(a, ndc, nsc, W1, b1r, wp1)
    a = agg128(h2, runs, cnts)
    feat2, t = _tc_layer_last(a, ndc, nsc, W2, b2r, wp2, sc0, sc1)

    sp = _make_score_agg()(t.reshape(NROW, 128), srcp, dstp)

    w80, m80 = _tc_select(sp[:NROW], sp[NROW:], nd80, bp11)
    return _tc_readout(w80.reshape(NPAD, 1), m80.reshape(NPAD, 1),
                       feat0, feat1, feat2, M0r, gr, br, M1)
